# baseline (speedup 1.0000x reference)
import functools

import jax
import jax.numpy as jnp
from jax import lax
from jax.experimental import pallas as pl
from jax.experimental.pallas import tpu as pltpu
from jax.experimental.pallas import tpu_sc as plsc

BATCH = 16384
EMBED_DIM = 16

_info = plsc.get_sparse_core_info()
_NC = _info.num_cores
_NS = _info.num_subcores
_NW = _NC * _NS
_BPW = BATCH // _NW

_mesh = plsc.VectorSubcoreMesh(core_axis_name="c", subcore_axis_name="s")


@functools.partial(
    pl.kernel,
    out_type=(
        jax.ShapeDtypeStruct((BATCH, EMBED_DIM), jnp.float32),
        jax.ShapeDtypeStruct((BATCH, EMBED_DIM), jnp.float32),
    ),
    mesh=_mesh,
    scratch_types=[
        pltpu.VMEM((_BPW,), jnp.int32),
        pltpu.VMEM((_BPW, EMBED_DIM), jnp.float32),
        pltpu.VMEM((_BPW,), jnp.int32),
        pltpu.VMEM((_BPW, EMBED_DIM), jnp.float32),
        pltpu.SemaphoreType.DMA,
        pltpu.SemaphoreType.DMA,
    ],
    compiler_params=pltpu.CompilerParams(use_tc_tiling_on_sc=False),
)
def _sc_gather2(uidx_hbm, iidx_hbm, utab_hbm, itab_hbm, uout_hbm, iout_hbm,
                uidx_v, urows_v, iidx_v, irows_v, usem, isem):
    wid = lax.axis_index("s") * _NC + lax.axis_index("c")
    base = wid * _BPW
    pltpu.sync_copy(uidx_hbm.at[pl.ds(base, _BPW)], uidx_v)
    pltpu.sync_copy(iidx_hbm.at[pl.ds(base, _BPW)], iidx_v)
    cu = pltpu.async_copy(utab_hbm.at[uidx_v], urows_v, usem)
    ci = pltpu.async_copy(itab_hbm.at[iidx_v], irows_v, isem)
    cu.wait()
    ci.wait()
    pltpu.sync_copy(urows_v, uout_hbm.at[pl.ds(base, _BPW)])
    pltpu.sync_copy(irows_v, iout_hbm.at[pl.ds(base, _BPW)])


def _mlp_body(u_ref, i_ref, w1u_ref, w1i_ref, b1_ref, w3_ref, b3_ref, o_ref):
    h = (jnp.dot(u_ref[...], w1u_ref[...], preferred_element_type=jnp.float32)
         + jnp.dot(i_ref[...], w1i_ref[...], preferred_element_type=jnp.float32)
         + b1_ref[...][None, :])
    h = jnp.maximum(h, 0.0)
    o = jnp.sum(h * w3_ref[...][None, :], axis=1) + b3_ref[...]
    o_ref[...] = jax.nn.sigmoid(o)


def _tc_mlp(u, i, w1u, w1i, b1, w3, b3):
    return pl.pallas_call(
        _mlp_body,
        out_shape=jax.ShapeDtypeStruct((BATCH,), jnp.float32),
    )(u, i, w1u, w1i, b1, w3, b3)


def kernel(user_indices, item_indices, emb_user, emb_item, W1, b1, W3, b3):
    uidx = user_indices.astype(jnp.int32)
    iidx = item_indices.astype(jnp.int32)
    u_rows, i_rows = _sc_gather2(uidx, iidx, emb_user, emb_item)
    w1u = W1[:EMBED_DIM]
    w1i = W1[EMBED_DIM:]
    w3 = W3[:, 0]
    return _tc_mlp(u_rows, i_rows, w1u, w1i, b1, w3, b3)


# speedup vs baseline: 1.0136x; 1.0136x over previous
import functools

import jax
import jax.numpy as jnp
from jax import lax
from jax.experimental import pallas as pl
from jax.experimental.pallas import tpu as pltpu
from jax.experimental.pallas import tpu_sc as plsc

BATCH = 16384
EMBED_DIM = 16
GROUP = 8
NROW = 1000000
NGRP = NROW // GROUP

_info = plsc.get_sparse_core_info()
_NC = _info.num_cores
_NS = _info.num_subcores
_NW = _NC * _NS
_BPW = BATCH // _NW
_CHUNK = 128
_NCHUNK = _BPW // _CHUNK

_mesh = plsc.VectorSubcoreMesh(core_axis_name="c", subcore_axis_name="s")


def _extract_chunk(rows_v, idx_v, out_v, c):
    lane = lax.iota(jnp.int32, 16)
    for t in range(_CHUNK // 16):
        idx16 = idx_v[pl.ds(c * _CHUNK + t * 16, 16)]
        rows = lane + t * 16
        cols = (idx16 & 7) * EMBED_DIM
        for d in range(EMBED_DIM):
            vals = plsc.load_gather(rows_v, [rows, cols + d])
            out_v[d, pl.ds(c * _CHUNK + t * 16, 16)] = vals


@functools.partial(
    pl.kernel,
    out_type=(
        jax.ShapeDtypeStruct((EMBED_DIM, BATCH), jnp.float32),
        jax.ShapeDtypeStruct((EMBED_DIM, BATCH), jnp.float32),
    ),
    mesh=_mesh,
    scratch_types=[
        pltpu.VMEM((_BPW,), jnp.int32),
        pltpu.VMEM((_BPW,), jnp.int32),
        pltpu.VMEM((_BPW,), jnp.int32),
        pltpu.VMEM((_BPW,), jnp.int32),
        pltpu.VMEM((2, _CHUNK, 128), jnp.float32),
        pltpu.VMEM((2, _CHUNK, 128), jnp.float32),
        pltpu.VMEM((EMBED_DIM, _BPW), jnp.float32),
        pltpu.VMEM((EMBED_DIM, _BPW), jnp.float32),
        pltpu.SemaphoreType.DMA,
        pltpu.SemaphoreType.DMA,
    ],
    compiler_params=pltpu.CompilerParams(needs_layout_passes=False),
)
def _sc_gather2(uidx_hbm, iidx_hbm, utab_hbm, itab_hbm, uout_hbm, iout_hbm,
                uidx_v, iidx_v, ug_v, ig_v, ubuf, ibuf, uout_v, iout_v,
                usem, isem):
    wid = lax.axis_index("s") * _NC + lax.axis_index("c")
    base = wid * _BPW
    pltpu.sync_copy(uidx_hbm.at[pl.ds(base, _BPW)], uidx_v)
    pltpu.sync_copy(iidx_hbm.at[pl.ds(base, _BPW)], iidx_v)
    for k in range(_BPW // 16):
        s = pl.ds(k * 16, 16)
        ug_v[s] = lax.shift_right_logical(uidx_v[s], 3)
        ig_v[s] = lax.shift_right_logical(iidx_v[s], 3)

    def fire(c):
        b = c % 2
        cu = pltpu.async_copy(utab_hbm.at[ug_v.at[pl.ds(c * _CHUNK, _CHUNK)]],
                              ubuf.at[b], usem)
        ci = pltpu.async_copy(itab_hbm.at[ig_v.at[pl.ds(c * _CHUNK, _CHUNK)]],
                              ibuf.at[b], isem)
        return cu, ci

    inflight = fire(0)
    for c in range(_NCHUNK):
        cu, ci = inflight
        if c + 1 < _NCHUNK:
            inflight = fire(c + 1)
        cu.wait()
        _extract_chunk(ubuf.at[c % 2], uidx_v, uout_v, c)
        ci.wait()
        _extract_chunk(ibuf.at[c % 2], iidx_v, iout_v, c)

    pltpu.sync_copy(uout_v, uout_hbm.at[:, pl.ds(base, _BPW)])
    pltpu.sync_copy(iout_v, iout_hbm.at[:, pl.ds(base, _BPW)])


def _mlp_body(ut_ref, it_ref, w1ut_ref, w1it_ref, b1_ref, w3_ref, b3_ref,
              o_ref):
    h = (jnp.dot(w1ut_ref[...], ut_ref[...],
                 preferred_element_type=jnp.float32)
         + jnp.dot(w1it_ref[...], it_ref[...],
                   preferred_element_type=jnp.float32)
         + b1_ref[...][:, None])
    h = jnp.maximum(h, 0.0)
    o = jnp.sum(h * w3_ref[...][:, None], axis=0) + b3_ref[...]
    o_ref[...] = jax.nn.sigmoid(o)


def _tc_mlp(ut, it, w1ut, w1it, b1, w3, b3):
    return pl.pallas_call(
        _mlp_body,
        out_shape=jax.ShapeDtypeStruct((BATCH,), jnp.float32),
    )(ut, it, w1ut, w1it, b1, w3, b3)


def kernel(user_indices, item_indices, emb_user, emb_item, W1, b1, W3, b3):
    uidx = user_indices.astype(jnp.int32)
    iidx = item_indices.astype(jnp.int32)
    utab = emb_user.reshape(NGRP, GROUP * EMBED_DIM)
    itab = emb_item.reshape(NGRP, GROUP * EMBED_DIM)
    u_t, i_t = _sc_gather2(uidx, iidx, utab, itab)
    w1ut = W1[:EMBED_DIM].T
    w1it = W1[EMBED_DIM:].T
    w3 = W3[:, 0]
    return _tc_mlp(u_t, i_t, w1ut, w1it, b1, w3, b3)
